# Initial kernel scaffold; baseline (speedup 1.0000x reference)
import jax
import jax.numpy as jnp
from jax.experimental import pallas as pl


def kernel(x, edge_index, W1l, W1r, b1, W2l, W2r, b2, Wfc, bfc):
    raise NotImplementedError("write your pallas kernel here")


# speedup vs baseline: 5.0115x; 5.0115x over previous
import jax
import jax.numpy as jnp
from jax import lax
from jax.experimental import pallas as pl
from jax.experimental.pallas import tpu as pltpu
from jax.experimental.pallas import tpu_sc as plsc

NC = 2
NS = 16
DEGW = 16


def _zero_fill(buf, rows, width):
  z16 = jnp.zeros((16,), jnp.float32)

  def zf(i, _):
    for k in range(width // 16):
      buf[i, pl.ds(k * 16, 16)] = z16
    return 0

  lax.fori_loop(0, rows, zf, 0)


def _sc_agg(n, d, e, c):
  rows_per_tile = n // NS
  e_per_core = e // NC
  e_per_tile = e_per_core // NS
  nch = e_per_tile // c
  zc = min(c, 128)
  nz = rows_per_tile // zc

  scratch = [
      pltpu.VMEM_SHARED((n, d), jnp.float32),
      pltpu.VMEM((c,), jnp.int32),
      pltpu.VMEM((c,), jnp.int32),
      pltpu.VMEM((c, d), jnp.float32),
      pltpu.SemaphoreType.DMA,
  ]
  mesh = plsc.VectorSubcoreMesh(core_axis_name="c", subcore_axis_name="s",
                                num_cores=NC, num_subcores=NS)

  def body(table_hbm, src_hbm, dst_hbm, psum_hbm,
           acc_sh, src_idx, dst_idx, rows_v, sem):
    core = lax.axis_index("c")
    sid = lax.axis_index("s")
    r0 = sid * rows_per_tile

    _zero_fill(rows_v, zc, d)
    for k in range(nz):
      pltpu.sync_copy(rows_v.at[pl.ds(0, zc)],
                      acc_sh.at[pl.ds(r0 + k * zc, zc)])
    plsc.subcore_barrier()

    e_base = core * e_per_core + sid * e_per_tile

    def step(j, _):
      off = e_base + j * c
      pltpu.sync_copy(src_hbm.at[pl.ds(off, c)], src_idx)
      pltpu.sync_copy(dst_hbm.at[pl.ds(off, c)], dst_idx)
      pltpu.async_copy(table_hbm.at[src_idx], rows_v, sem).wait()
      pltpu.sync_copy(rows_v, acc_sh.at[dst_idx], add=True)
      return 0

    lax.fori_loop(0, nch, step, 0)
    plsc.subcore_barrier()

    pltpu.sync_copy(acc_sh.at[pl.ds(r0, rows_per_tile)],
                    psum_hbm.at[core, pl.ds(r0, rows_per_tile)])

  return pl.kernel(
      body,
      out_type=jax.ShapeDtypeStruct((NC, n, d), jnp.float32),
      mesh=mesh,
      scratch_types=scratch,
  )


def _sc_deg(n, e, c, degw=DEGW, tc_tiling=None):
  rows_per_tile = n // NS
  e_per_core = e // NC
  e_per_tile = e_per_core // NS
  nch = e_per_tile // c
  zc = min(c, 128)
  nz = rows_per_tile // zc

  scratch = [
      pltpu.VMEM_SHARED((n, degw), jnp.float32),
      pltpu.VMEM((c,), jnp.int32),
      pltpu.VMEM((c, degw), jnp.float32),
  ]
  mesh = plsc.VectorSubcoreMesh(core_axis_name="c", subcore_axis_name="s",
                                num_cores=NC, num_subcores=NS)

  def body(dst_hbm, degp_hbm, deg_sh, dst_idx, ones_v):
    core = lax.axis_index("c")
    sid = lax.axis_index("s")
    r0 = sid * rows_per_tile

    _zero_fill(ones_v, zc, degw)
    for k in range(nz):
      pltpu.sync_copy(ones_v.at[pl.ds(0, zc)],
                      deg_sh.at[pl.ds(r0 + k * zc, zc)])
    o16 = jnp.ones((16,), jnp.float32)

    def ofill(i, _):
      for k in range(degw // 16):
        ones_v[i, pl.ds(k * 16, 16)] = o16
      return 0

    lax.fori_loop(0, c, ofill, 0)
    plsc.subcore_barrier()

    e_base = core * e_per_core + sid * e_per_tile

    def step(j, _):
      off = e_base + j * c
      pltpu.sync_copy(dst_hbm.at[pl.ds(off, c)], dst_idx)
      pltpu.sync_copy(ones_v, deg_sh.at[dst_idx], add=True)
      return 0

    lax.fori_loop(0, nch, step, 0)
    plsc.subcore_barrier()

    pltpu.sync_copy(deg_sh.at[pl.ds(r0, rows_per_tile)],
                    degp_hbm.at[core, pl.ds(r0, rows_per_tile)])

  return pl.kernel(
      body,
      out_type=jax.ShapeDtypeStruct((NC, n, degw), jnp.float32),
      mesh=mesh,
      scratch_types=scratch,
      compiler_params=(None if tc_tiling is None
                       else pltpu.CompilerParams(use_tc_tiling_on_sc=tc_tiling)),
  )


def _tc_layer(n, d, bm, final):
  def body(*refs):
    if final:
      p0, p1, g0, g1, h, wl, wr, b, wf, bf, o = refs
    else:
      p0, p1, g0, g1, h, wl, wr, b, o = refs
    deg = g0[:, 0:1] + g1[:, 0:1]
    rdeg = 1.0 / jnp.maximum(deg, 1.0)
    agg = (p0[:] + p1[:]) * rdeg
    z = (jnp.dot(agg, wl[:], preferred_element_type=jnp.float32)
         + jnp.dot(h[:], wr[:], preferred_element_type=jnp.float32) + b[:])
    z = jnp.maximum(z, 0.0)
    if final:
      o[:] = jnp.dot(z, wf[:], preferred_element_type=jnp.float32) + bf[:]
    else:
      o[:] = z

  grid = (n // bm,)
  row = lambda i: (i, 0)
  fixed = lambda i: (0, 0)
  in_specs = [
      pl.BlockSpec((bm, d), row),
      pl.BlockSpec((bm, d), row),
      pl.BlockSpec((bm, DEGW), row),
      pl.BlockSpec((bm, DEGW), row),
      pl.BlockSpec((bm, d), row),
      pl.BlockSpec((d, d), fixed),
      pl.BlockSpec((d, d), fixed),
      pl.BlockSpec((1, d), fixed),
  ]
  if final:
    in_specs += [pl.BlockSpec((d, d), fixed), pl.BlockSpec((1, d), fixed)]
  return pl.pallas_call(
      body,
      grid=grid,
      in_specs=in_specs,
      out_specs=pl.BlockSpec((bm, d), row),
      out_shape=jax.ShapeDtypeStruct((n, d), jnp.float32),
  )


def kernel(x, edge_index, W1l, W1r, b1, W2l, W2r, b2, Wfc, bfc):
  n, d = x.shape
  e = edge_index.shape[1]
  src = edge_index[0]
  dst = edge_index[1]

  npad = ((n + NS * 64 - 1) // (NS * 64)) * (NS * 64)
  xp = jnp.zeros((npad, d), x.dtype).at[:n].set(x) if npad != n else x

  c = 80
  sc_agg = _sc_agg(npad, d, e, c)
  sc_deg = _sc_deg(npad, e, c, degw=DEGW, tc_tiling=False)
  tc1 = _tc_layer(npad, d, bm=1024, final=False)
  tc2 = _tc_layer(npad, d, bm=1024, final=True)

  degp = sc_deg(dst)
  psum1 = sc_agg(xp, src, dst)
  h1 = tc1(psum1[0], psum1[1], degp[0], degp[1], xp, W1l, W1r,
           b1.reshape(1, d))
  psum2 = sc_agg(h1, src, dst)
  wf = jnp.zeros((d, d), jnp.float32).at[:, : Wfc.shape[1]].set(Wfc)
  bf = jnp.zeros((1, d), jnp.float32).at[0, : bfc.shape[0]].set(bfc)
  outp = tc2(psum2[0], psum2[1], degp[0], degp[1], h1, W2l, W2r,
             b2.reshape(1, d), wf, bf)
  return outp[:n, : Wfc.shape[1]]
